# baseline (speedup 1.0000x reference)
import jax
import jax.numpy as jnp
from jax.experimental import pallas as pl
from jax.experimental.pallas import tpu as pltpu

_NBUF = 4


def _make_kernel(bm, nblk):
    def _geo_conv_kernel(x_ref, adj_ref, w_ref, o_ref, buf, sem):
        i = pl.program_id(0)

        def start(b):
            slot = jax.lax.rem(b, _NBUF)
            pltpu.make_async_copy(
                adj_ref.at[pl.ds(b * bm, bm), :],
                buf.at[slot],
                sem.at[slot],
            ).start()

        @pl.when(i == 0)
        def _():
            for k in range(min(_NBUF, nblk)):
                start(k)

        @pl.when(jnp.logical_and(i > 0, i + _NBUF - 1 < nblk))
        def _():
            start(i + _NBUF - 1)

        slot = jax.lax.rem(i, _NBUF)
        pltpu.make_async_copy(
            adj_ref.at[pl.ds(i * bm, bm), :],
            buf.at[slot],
            sem.at[slot],
        ).wait()
        y = jax.lax.dot_general(
            buf[slot], x_ref[...],
            dimension_numbers=(((1,), (0,)), ((), ())),
            preferred_element_type=jnp.float32,
        )
        o_ref[...] = 0.75 * y + jax.lax.dot_general(
            y, w_ref[...],
            dimension_numbers=(((1,), (1,)), ((), ())),
            preferred_element_type=jnp.float32,
        )

    return _geo_conv_kernel


def kernel(x, adj, weight):
    n, d = x.shape
    bm = 200 if n % 200 == 0 else n
    nblk = n // bm
    return pl.pallas_call(
        _make_kernel(bm, nblk),
        grid=(nblk,),
        in_specs=[
            pl.BlockSpec((n, d), lambda i: (0, 0)),
            pl.BlockSpec(memory_space=pl.ANY),
            pl.BlockSpec((d, d), lambda i: (0, 0)),
        ],
        out_specs=pl.BlockSpec((bm, d), lambda i: (i, 0)),
        out_shape=jax.ShapeDtypeStruct((n, d), jnp.float32),
        scratch_shapes=[
            pltpu.VMEM((_NBUF, bm, n), jnp.float32),
            pltpu.SemaphoreType.DMA((_NBUF,)),
        ],
        compiler_params=pltpu.CompilerParams(
            dimension_semantics=("arbitrary",),
        ),
    )(x, adj, weight)
